# baseline (device time: 355777 ns/iter reference)
import jax
import jax.numpy as jnp
from jax import lax
from jax.experimental import pallas as pl
from jax.experimental.pallas import tpu as pltpu

N_DEV = 4


def _ring_allgather(x):
    m_per, k = x.shape
    M = N_DEV * m_per

    def body(x_hbm, out_hbm, comm_ref, send_sems, recv_sems, local_sem):
        my = lax.axis_index("i")
        left = lax.rem(my + N_DEV - 1, N_DEV)
        right = lax.rem(my + 1, N_DEV)

        barrier_sem = pltpu.get_barrier_semaphore()
        for nbr in [left, right]:
            pl.semaphore_signal(
                barrier_sem, inc=1,
                device_id=(nbr,), device_id_type=pl.DeviceIdType.MESH,
            )
        pl.semaphore_wait(barrier_sem, 2)

        cp = pltpu.make_async_copy(x_hbm, comm_ref.at[0], local_sem)
        cp.start()
        cp.wait()
        cp = pltpu.make_async_copy(
            comm_ref.at[0], out_hbm.at[pl.ds(my * m_per, m_per)], local_sem
        )
        cp.start()
        cp.wait()

        for h in range(N_DEV - 1):
            s, r = h % 2, (h + 1) % 2
            rdma = pltpu.make_async_remote_copy(
                src_ref=comm_ref.at[s],
                dst_ref=comm_ref.at[r],
                send_sem=send_sems.at[s],
                recv_sem=recv_sems.at[r],
                device_id=(right,),
                device_id_type=pl.DeviceIdType.MESH,
            )
            rdma.start()
            rdma.wait()
            origin = lax.rem(my + N_DEV - 1 - h, N_DEV)
            cp = pltpu.make_async_copy(
                comm_ref.at[r], out_hbm.at[pl.ds(origin * m_per, m_per)], local_sem
            )
            cp.start()
            cp.wait()

    return pl.pallas_call(
        body,
        out_shape=jax.ShapeDtypeStruct((M, k), x.dtype),
        in_specs=[pl.BlockSpec(memory_space=pl.ANY)],
        out_specs=pl.BlockSpec(memory_space=pl.ANY),
        scratch_shapes=[
            pltpu.VMEM((2, m_per, k), x.dtype),
            pltpu.SemaphoreType.DMA((2,)),
            pltpu.SemaphoreType.DMA((2,)),
            pltpu.SemaphoreType.DMA,
        ],
        compiler_params=pltpu.CompilerParams(collective_id=0),
    )(x)


def _gemm_relu_amax(x_full, w):
    M, K = x_full.shape
    _, n_per = w.shape
    bm, bn = 512, 512

    def body(x_ref, w_ref, y_ref, amax_out, acc_ref):
        i, j = pl.program_id(0), pl.program_id(1)

        @pl.when((i == 0) & (j == 0))
        def _():
            acc_ref[0, 0] = 0.0

        z = jnp.dot(x_ref[...], w_ref[...], preferred_element_type=jnp.float32)
        z = jnp.maximum(z, 0.0)
        y_ref[...] = z
        acc_ref[0, 0] = jnp.maximum(acc_ref[0, 0], jnp.max(z))
        amax_out[0, 0] = acc_ref[0, 0]

    return pl.pallas_call(
        body,
        grid=(M // bm, n_per // bn),
        in_specs=[
            pl.BlockSpec((bm, K), lambda i, j: (i, 0)),
            pl.BlockSpec((K, bn), lambda i, j: (0, j)),
        ],
        out_specs=[
            pl.BlockSpec((bm, bn), lambda i, j: (i, j)),
            pl.BlockSpec((1, 1), lambda i, j: (0, 0), memory_space=pltpu.SMEM),
        ],
        out_shape=[
            jax.ShapeDtypeStruct((M, n_per), jnp.float32),
            jax.ShapeDtypeStruct((1, 1), jnp.float32),
        ],
        scratch_shapes=[pltpu.SMEM((1, 1), jnp.float32)],
        compiler_params=pltpu.CompilerParams(vmem_limit_bytes=58 * 2**20),
    )(x_full, w)


def _ag_gemm_fused(x, w):
    m_per, K = x.shape
    _, n_per = w.shape
    M = N_DEV * m_per
    half = m_per // 2
    bn = 512
    n_tiles = n_per // bn

    def body(x_hbm, w_hbm, y_hbm, amax_out,
             cw_ref, ccw_ref, wt_ref, ys_ref,
             cw_send, cw_recv, ccw_send, ccw_recv,
             credit_cw, credit_ccw, wsem, ysem, xsem):
        my = lax.axis_index("i")
        left = lax.rem(my + N_DEV - 1, N_DEV)
        right = lax.rem(my + 1, N_DEV)

        cp_top = pltpu.make_async_copy(
            x_hbm.at[pl.ds(0, half)], cw_ref.at[0], xsem.at[0])
        cp_bot = pltpu.make_async_copy(
            x_hbm.at[pl.ds(half, half)], ccw_ref.at[0], xsem.at[1])
        cp_top.start()
        cp_bot.start()

        wt_pending = {}

        def load_wtile(j, slot):
            cp = pltpu.make_async_copy(
                w_hbm.at[:, pl.ds(j * bn, bn)], wt_ref.at[slot],
                wsem.at[slot])
            cp.start()
            wt_pending[slot] = cp

        load_wtile(0, 0)

        barrier_sem = pltpu.get_barrier_semaphore()
        for nbr in [left, right]:
            pl.semaphore_signal(
                barrier_sem, inc=1,
                device_id=(nbr,), device_id_type=pl.DeviceIdType.MESH,
            )
        pl.semaphore_wait(barrier_sem, 2)

        state = {"amax": jnp.float32(0.0), "store": 0}
        ys_pending = [None, None]

        def store_y(z, row, j, nrows):
            slot = state["store"] % 2
            state["store"] += 1
            if ys_pending[slot] is not None:
                ys_pending[slot].wait()
            ys_ref[slot, :nrows] = z
            cp = pltpu.make_async_copy(
                ys_ref.at[slot, pl.ds(0, nrows)],
                y_hbm.at[pl.ds(row, nrows), pl.ds(j * bn, bn)],
                ysem.at[slot])
            cp.start()
            ys_pending[slot] = cp

        def compute_phase(h, row_off=0, nrows=half, prefetch_next=True):
            s = h % 2
            row_a = lax.rem(my + N_DEV - h, N_DEV) * m_per + row_off
            row_b = lax.rem(my + h, N_DEV) * m_per + half + row_off
            for j in range(n_tiles):
                slot = j % 2
                wt_pending.pop(slot).wait()
                if j + 1 < n_tiles:
                    load_wtile(j + 1, 1 - slot)
                elif prefetch_next:
                    load_wtile(0, 1 - slot)
                za = jnp.dot(cw_ref[s, row_off:row_off + nrows],
                             wt_ref[slot],
                             preferred_element_type=jnp.float32)
                za = jnp.maximum(za, 0.0)
                state["amax"] = jnp.maximum(state["amax"], jnp.max(za))
                store_y(za, row_a, j, nrows)
                zb = jnp.dot(ccw_ref[s, row_off:row_off + nrows],
                             wt_ref[slot],
                             preferred_element_type=jnp.float32)
                zb = jnp.maximum(zb, 0.0)
                state["amax"] = jnp.maximum(state["amax"], jnp.max(zb))
                store_y(zb, row_b, j, nrows)

        qh = half // 2

        def make_rdma(ref, s, r, row_off, nrows, sem_idx, send, recv, tgt):
            return pltpu.make_async_remote_copy(
                src_ref=ref.at[s, pl.ds(row_off, nrows)],
                dst_ref=ref.at[r, pl.ds(row_off, nrows)],
                send_sem=send.at[sem_idx], recv_sem=recv.at[sem_idx],
                device_id=(tgt,), device_id_type=pl.DeviceIdType.MESH)

        for h in range(N_DEV - 2):
            s, r = h % 2, (h + 1) % 2
            if h > 0:
                pl.semaphore_wait(credit_cw, 1)
                pl.semaphore_wait(credit_ccw, 1)
                rdma_cw = make_rdma(cw_ref, s, r, 0, half, h,
                                    cw_send, cw_recv, right)
                rdma_ccw = make_rdma(ccw_ref, s, r, 0, half, h,
                                     ccw_send, ccw_recv, left)
            else:
                rdma_cw = pltpu.make_async_remote_copy(
                    src_ref=x_hbm.at[pl.ds(0, half)],
                    dst_ref=cw_ref.at[1, pl.ds(0, half)],
                    send_sem=cw_send.at[0], recv_sem=cw_recv.at[0],
                    device_id=(right,), device_id_type=pl.DeviceIdType.MESH)
                rdma_ccw = pltpu.make_async_remote_copy(
                    src_ref=x_hbm.at[pl.ds(half, half)],
                    dst_ref=ccw_ref.at[1, pl.ds(0, half)],
                    send_sem=ccw_send.at[0], recv_sem=ccw_recv.at[0],
                    device_id=(left,), device_id_type=pl.DeviceIdType.MESH)
            rdma_cw.start()
            rdma_ccw.start()
            if h == 0:
                cp_top.wait()
                cp_bot.wait()
            compute_phase(h)
            rdma_cw.wait_send()
            rdma_ccw.wait_send()
            pl.semaphore_signal(
                credit_cw, inc=1,
                device_id=(left,), device_id_type=pl.DeviceIdType.MESH)
            pl.semaphore_signal(
                credit_ccw, inc=1,
                device_id=(right,), device_id_type=pl.DeviceIdType.MESH)
            rdma_cw.wait_recv()
            rdma_ccw.wait_recv()

        h = N_DEV - 2
        s, r = h % 2, (h + 1) % 2
        pl.semaphore_wait(credit_cw, 1)
        pl.semaphore_wait(credit_ccw, 1)
        subs = []
        for p in range(2):
            sub_cw = make_rdma(cw_ref, s, r, p * qh, qh, h + p,
                               cw_send, cw_recv, right)
            sub_ccw = make_rdma(ccw_ref, s, r, p * qh, qh, h + p,
                                ccw_send, ccw_recv, left)
            sub_cw.start()
            sub_ccw.start()
            subs.append((sub_cw, sub_ccw))
        compute_phase(h)
        for sub_cw, sub_ccw in subs:
            sub_cw.wait_send()
            sub_ccw.wait_send()
        for p in range(2):
            sub_cw, sub_ccw = subs[p]
            sub_cw.wait_recv()
            sub_ccw.wait_recv()
            compute_phase(N_DEV - 1, row_off=p * qh, nrows=qh,
                          prefetch_next=(p == 0))

        amax_out[0, 0] = state["amax"]
        for cp in ys_pending:
            if cp is not None:
                cp.wait()

    return pl.pallas_call(
        body,
        out_shape=[
            jax.ShapeDtypeStruct((M, n_per), jnp.float32),
            jax.ShapeDtypeStruct((1, 1), jnp.float32),
        ],
        in_specs=[
            pl.BlockSpec(memory_space=pl.ANY),
            pl.BlockSpec(memory_space=pl.ANY),
        ],
        out_specs=[
            pl.BlockSpec(memory_space=pl.ANY),
            pl.BlockSpec(memory_space=pltpu.SMEM),
        ],
        scratch_shapes=[
            pltpu.VMEM((2, half, K), jnp.float32),
            pltpu.VMEM((2, half, K), jnp.float32),
            pltpu.VMEM((2, K, bn), jnp.float32),
            pltpu.VMEM((2, half, bn), jnp.float32),
            pltpu.SemaphoreType.DMA((N_DEV,)),
            pltpu.SemaphoreType.DMA((N_DEV,)),
            pltpu.SemaphoreType.DMA((N_DEV,)),
            pltpu.SemaphoreType.DMA((N_DEV,)),
            pltpu.SemaphoreType.REGULAR,
            pltpu.SemaphoreType.REGULAR,
            pltpu.SemaphoreType.DMA((2,)),
            pltpu.SemaphoreType.DMA((2,)),
            pltpu.SemaphoreType.DMA((2,)),
        ],
        compiler_params=pltpu.CompilerParams(
            collective_id=0, vmem_limit_bytes=60 * 2**20),
    )(x, w)


def _global_amax(local_amax):

    def body(a_ref, g_ref, buf_ref, send_sems, recv_sems):
        my = lax.axis_index("i")

        barrier_sem = pltpu.get_barrier_semaphore()
        for off in range(1, N_DEV):
            tgt = lax.rem(my + off, N_DEV)
            pl.semaphore_signal(
                barrier_sem, inc=1,
                device_id=(tgt,), device_id_type=pl.DeviceIdType.MESH,
            )
        pl.semaphore_wait(barrier_sem, N_DEV - 1)

        buf_ref[my] = jnp.full((8, 128), a_ref[0, 0], jnp.float32)

        sends = []
        for off in range(1, N_DEV):
            tgt = lax.rem(my + off, N_DEV)
            rdma = pltpu.make_async_remote_copy(
                src_ref=buf_ref.at[my],
                dst_ref=buf_ref.at[my],
                send_sem=send_sems.at[off - 1],
                recv_sem=recv_sems.at[off - 1],
                device_id=(tgt,),
                device_id_type=pl.DeviceIdType.MESH,
            )
            rdma.start()
            sends.append(rdma)

        for off in range(1, N_DEV):
            src = lax.rem(my + N_DEV - off, N_DEV)
            recv = pltpu.make_async_remote_copy(
                src_ref=buf_ref.at[my],
                dst_ref=buf_ref.at[src],
                send_sem=send_sems.at[off - 1],
                recv_sem=recv_sems.at[off - 1],
                device_id=(src,),
                device_id_type=pl.DeviceIdType.MESH,
            )
            recv.wait_recv()
        for rdma in sends:
            rdma.wait_send()

        g_ref[0, 0] = jnp.max(buf_ref[...])

    return pl.pallas_call(
        body,
        out_shape=jax.ShapeDtypeStruct((1, 1), jnp.float32),
        in_specs=[pl.BlockSpec(memory_space=pltpu.SMEM)],
        out_specs=pl.BlockSpec(memory_space=pltpu.SMEM),
        scratch_shapes=[
            pltpu.VMEM((N_DEV, 8, 128), jnp.float32),
            pltpu.SemaphoreType.DMA((N_DEV - 1,)),
            pltpu.SemaphoreType.DMA((N_DEV - 1,)),
        ],
        compiler_params=pltpu.CompilerParams(collective_id=1),
    )(local_amax)


def _quant_epilogue(y, gmax):
    M, n_per = y.shape
    bm = 1024

    def body(g_ref, y_ref, o_ref):
        g = jnp.maximum(g_ref[0, 0], 1e-30)
        scale = g / 448.0
        inv = 448.0 / g
        q = (y_ref[...] * inv).astype(jnp.float8_e4m3fn)
        o_ref[...] = q.astype(jnp.float32) * scale

    return pl.pallas_call(
        body,
        grid=(M // bm,),
        in_specs=[
            pl.BlockSpec((1, 1), lambda i: (0, 0), memory_space=pltpu.SMEM),
            pl.BlockSpec((bm, n_per), lambda i: (i, 0)),
        ],
        out_specs=pl.BlockSpec((bm, n_per), lambda i: (i, 0)),
        out_shape=jax.ShapeDtypeStruct((M, n_per), jnp.float32),
        compiler_params=pltpu.CompilerParams(vmem_limit_bytes=58 * 2**20),
    )(gmax, y)


def _exchange_quant(y, local_amax):
    M, n_per = y.shape
    bm = 1024

    def body(a_ref, y_ref, o_ref, scale_ref, buf_ref, send_sems, recv_sems):
        i = pl.program_id(0)

        @pl.when(i == 0)
        def _():
            my = lax.axis_index("i")

            barrier_sem = pltpu.get_barrier_semaphore()
            for off in range(1, N_DEV):
                tgt = lax.rem(my + off, N_DEV)
                pl.semaphore_signal(
                    barrier_sem, inc=1,
                    device_id=(tgt,), device_id_type=pl.DeviceIdType.MESH,
                )
            pl.semaphore_wait(barrier_sem, N_DEV - 1)

            buf_ref[my] = jnp.full((8, 128), a_ref[0, 0], jnp.float32)

            sends = []
            for off in range(1, N_DEV):
                tgt = lax.rem(my + off, N_DEV)
                rdma = pltpu.make_async_remote_copy(
                    src_ref=buf_ref.at[my],
                    dst_ref=buf_ref.at[my],
                    send_sem=send_sems.at[off - 1],
                    recv_sem=recv_sems.at[off - 1],
                    device_id=(tgt,),
                    device_id_type=pl.DeviceIdType.MESH,
                )
                rdma.start()
                sends.append(rdma)

            for off in range(1, N_DEV):
                src = lax.rem(my + N_DEV - off, N_DEV)
                recv = pltpu.make_async_remote_copy(
                    src_ref=buf_ref.at[my],
                    dst_ref=buf_ref.at[src],
                    send_sem=send_sems.at[off - 1],
                    recv_sem=recv_sems.at[off - 1],
                    device_id=(src,),
                    device_id_type=pl.DeviceIdType.MESH,
                )
                recv.wait_recv()
            for rdma in sends:
                rdma.wait_send()

            g = jnp.maximum(jnp.max(buf_ref[...]), 1e-30)
            scale_ref[0] = g / 448.0
            scale_ref[1] = 448.0 / g

        q = (y_ref[...] * scale_ref[1]).astype(jnp.float8_e4m3fn)
        o_ref[...] = q.astype(jnp.float32) * scale_ref[0]

    return pl.pallas_call(
        body,
        grid=(M // bm,),
        in_specs=[
            pl.BlockSpec((1, 1), lambda i: (0, 0), memory_space=pltpu.SMEM),
            pl.BlockSpec((bm, n_per), lambda i: (i, 0)),
        ],
        out_specs=pl.BlockSpec((bm, n_per), lambda i: (i, 0)),
        out_shape=jax.ShapeDtypeStruct((M, n_per), jnp.float32),
        scratch_shapes=[
            pltpu.SMEM((2,), jnp.float32),
            pltpu.VMEM((N_DEV, 8, 128), jnp.float32),
            pltpu.SemaphoreType.DMA((N_DEV - 1,)),
            pltpu.SemaphoreType.DMA((N_DEV - 1,)),
        ],
        compiler_params=pltpu.CompilerParams(
            collective_id=1, vmem_limit_bytes=58 * 2**20),
    )(local_amax, y)


def kernel(x, w_mat):
    y, local_amax = _ag_gemm_fused(x, w_mat)
    return _exchange_quant(y, local_amax)


# device time: 322472 ns/iter; 1.1033x vs baseline; 1.1033x over previous
import jax
import jax.numpy as jnp
from jax import lax
from jax.experimental import pallas as pl
from jax.experimental.pallas import tpu as pltpu

N_DEV = 4


def _ring_allgather(x):
    m_per, k = x.shape
    M = N_DEV * m_per

    def body(x_hbm, out_hbm, comm_ref, send_sems, recv_sems, local_sem):
        my = lax.axis_index("i")
        left = lax.rem(my + N_DEV - 1, N_DEV)
        right = lax.rem(my + 1, N_DEV)

        barrier_sem = pltpu.get_barrier_semaphore()
        for nbr in [left, right]:
            pl.semaphore_signal(
                barrier_sem, inc=1,
                device_id=(nbr,), device_id_type=pl.DeviceIdType.MESH,
            )
        pl.semaphore_wait(barrier_sem, 2)

        cp = pltpu.make_async_copy(x_hbm, comm_ref.at[0], local_sem)
        cp.start()
        cp.wait()
        cp = pltpu.make_async_copy(
            comm_ref.at[0], out_hbm.at[pl.ds(my * m_per, m_per)], local_sem
        )
        cp.start()
        cp.wait()

        for h in range(N_DEV - 1):
            s, r = h % 2, (h + 1) % 2
            rdma = pltpu.make_async_remote_copy(
                src_ref=comm_ref.at[s],
                dst_ref=comm_ref.at[r],
                send_sem=send_sems.at[s],
                recv_sem=recv_sems.at[r],
                device_id=(right,),
                device_id_type=pl.DeviceIdType.MESH,
            )
            rdma.start()
            rdma.wait()
            origin = lax.rem(my + N_DEV - 1 - h, N_DEV)
            cp = pltpu.make_async_copy(
                comm_ref.at[r], out_hbm.at[pl.ds(origin * m_per, m_per)], local_sem
            )
            cp.start()
            cp.wait()

    return pl.pallas_call(
        body,
        out_shape=jax.ShapeDtypeStruct((M, k), x.dtype),
        in_specs=[pl.BlockSpec(memory_space=pl.ANY)],
        out_specs=pl.BlockSpec(memory_space=pl.ANY),
        scratch_shapes=[
            pltpu.VMEM((2, m_per, k), x.dtype),
            pltpu.SemaphoreType.DMA((2,)),
            pltpu.SemaphoreType.DMA((2,)),
            pltpu.SemaphoreType.DMA,
        ],
        compiler_params=pltpu.CompilerParams(collective_id=0),
    )(x)


def _gemm_relu_amax(x_full, w):
    M, K = x_full.shape
    _, n_per = w.shape
    bm, bn = 512, 512

    def body(x_ref, w_ref, y_ref, amax_out, acc_ref):
        i, j = pl.program_id(0), pl.program_id(1)

        @pl.when((i == 0) & (j == 0))
        def _():
            acc_ref[0, 0] = 0.0

        z = jnp.dot(x_ref[...], w_ref[...], preferred_element_type=jnp.float32)
        z = jnp.maximum(z, 0.0)
        y_ref[...] = z
        acc_ref[0, 0] = jnp.maximum(acc_ref[0, 0], jnp.max(z))
        amax_out[0, 0] = acc_ref[0, 0]

    return pl.pallas_call(
        body,
        grid=(M // bm, n_per // bn),
        in_specs=[
            pl.BlockSpec((bm, K), lambda i, j: (i, 0)),
            pl.BlockSpec((K, bn), lambda i, j: (0, j)),
        ],
        out_specs=[
            pl.BlockSpec((bm, bn), lambda i, j: (i, j)),
            pl.BlockSpec((1, 1), lambda i, j: (0, 0), memory_space=pltpu.SMEM),
        ],
        out_shape=[
            jax.ShapeDtypeStruct((M, n_per), jnp.float32),
            jax.ShapeDtypeStruct((1, 1), jnp.float32),
        ],
        scratch_shapes=[pltpu.SMEM((1, 1), jnp.float32)],
        compiler_params=pltpu.CompilerParams(vmem_limit_bytes=58 * 2**20),
    )(x_full, w)


def _ag_gemm_fused(x, w):
    m_per, K = x.shape
    _, n_per = w.shape
    M = N_DEV * m_per
    half = m_per // 2
    bn = 512
    n_tiles = n_per // bn

    def body(x_hbm, w_hbm, y_hbm, amax_out,
             cw_ref, ccw_ref, wt_ref, ys_ref,
             cw_send, cw_recv, ccw_send, ccw_recv,
             credit_cw, credit_ccw, wsem, ysem, xsem):
        my = lax.axis_index("i")
        left = lax.rem(my + N_DEV - 1, N_DEV)
        right = lax.rem(my + 1, N_DEV)

        cp_top = pltpu.make_async_copy(
            x_hbm.at[pl.ds(0, half)], cw_ref.at[0], xsem.at[0])
        cp_bot = pltpu.make_async_copy(
            x_hbm.at[pl.ds(half, half)], ccw_ref.at[0], xsem.at[1])
        cp_top.start()
        cp_bot.start()

        wt_pending = {}

        def load_wtile(j, slot):
            cp = pltpu.make_async_copy(
                w_hbm.at[:, pl.ds(j * bn, bn)], wt_ref.at[slot],
                wsem.at[slot])
            cp.start()
            wt_pending[slot] = cp

        load_wtile(0, 0)

        barrier_sem = pltpu.get_barrier_semaphore()
        for nbr in [left, right]:
            pl.semaphore_signal(
                barrier_sem, inc=1,
                device_id=(nbr,), device_id_type=pl.DeviceIdType.MESH,
            )
        pl.semaphore_wait(barrier_sem, 2)

        state = {"amax": jnp.float32(0.0), "store": 0}
        ys_pending = [None, None]

        def store_y(z, row, j, nrows):
            slot = state["store"] % 2
            state["store"] += 1
            if ys_pending[slot] is not None:
                ys_pending[slot].wait()
            ys_ref[slot, :nrows] = z
            cp = pltpu.make_async_copy(
                ys_ref.at[slot, pl.ds(0, nrows)],
                y_hbm.at[pl.ds(row, nrows), pl.ds(j * bn, bn)],
                ysem.at[slot])
            cp.start()
            ys_pending[slot] = cp

        def compute_phase(h, row_off=0, nrows=half, prefetch_next=True):
            s = h % 2
            row_a = lax.rem(my + N_DEV - h, N_DEV) * m_per + row_off
            row_b = lax.rem(my + h, N_DEV) * m_per + half + row_off
            for j in range(n_tiles):
                slot = j % 2
                wt_pending.pop(slot).wait()
                if j + 1 < n_tiles:
                    load_wtile(j + 1, 1 - slot)
                elif prefetch_next:
                    load_wtile(0, 1 - slot)
                za = jnp.dot(cw_ref[s, row_off:row_off + nrows],
                             wt_ref[slot],
                             preferred_element_type=jnp.float32)
                za = jnp.maximum(za, 0.0)
                state["amax"] = jnp.maximum(state["amax"], jnp.max(za))
                store_y(za, row_a, j, nrows)
                zb = jnp.dot(ccw_ref[s, row_off:row_off + nrows],
                             wt_ref[slot],
                             preferred_element_type=jnp.float32)
                zb = jnp.maximum(zb, 0.0)
                state["amax"] = jnp.maximum(state["amax"], jnp.max(zb))
                store_y(zb, row_b, j, nrows)

        qh = half // 2

        def make_rdma(ref, s, r, row_off, nrows, sem_idx, send, recv, tgt):
            return pltpu.make_async_remote_copy(
                src_ref=ref.at[s, pl.ds(row_off, nrows)],
                dst_ref=ref.at[r, pl.ds(row_off, nrows)],
                send_sem=send.at[sem_idx], recv_sem=recv.at[sem_idx],
                device_id=(tgt,), device_id_type=pl.DeviceIdType.MESH)

        for h in range(N_DEV - 2):
            s, r = h % 2, (h + 1) % 2
            if h > 0:
                pl.semaphore_wait(credit_cw, 1)
                pl.semaphore_wait(credit_ccw, 1)
                rdma_cw = make_rdma(cw_ref, s, r, 0, half, h,
                                    cw_send, cw_recv, right)
                rdma_ccw = make_rdma(ccw_ref, s, r, 0, half, h,
                                     ccw_send, ccw_recv, left)
            else:
                rdma_cw = pltpu.make_async_remote_copy(
                    src_ref=x_hbm.at[pl.ds(0, half)],
                    dst_ref=cw_ref.at[1, pl.ds(0, half)],
                    send_sem=cw_send.at[0], recv_sem=cw_recv.at[0],
                    device_id=(right,), device_id_type=pl.DeviceIdType.MESH)
                rdma_ccw = pltpu.make_async_remote_copy(
                    src_ref=x_hbm.at[pl.ds(half, half)],
                    dst_ref=ccw_ref.at[1, pl.ds(0, half)],
                    send_sem=ccw_send.at[0], recv_sem=ccw_recv.at[0],
                    device_id=(left,), device_id_type=pl.DeviceIdType.MESH)
            rdma_cw.start()
            rdma_ccw.start()
            if h == 0:
                cp_top.wait()
                cp_bot.wait()
            compute_phase(h)
            rdma_cw.wait_send()
            rdma_ccw.wait_send()
            pl.semaphore_signal(
                credit_cw, inc=1,
                device_id=(left,), device_id_type=pl.DeviceIdType.MESH)
            pl.semaphore_signal(
                credit_ccw, inc=1,
                device_id=(right,), device_id_type=pl.DeviceIdType.MESH)
            rdma_cw.wait_recv()
            rdma_ccw.wait_recv()

        h = N_DEV - 2
        s, r = h % 2, (h + 1) % 2
        pl.semaphore_wait(credit_cw, 1)
        pl.semaphore_wait(credit_ccw, 1)
        subs = []
        for p in range(2):
            sub_cw = make_rdma(cw_ref, s, r, p * qh, qh, h + p,
                               cw_send, cw_recv, right)
            sub_ccw = make_rdma(ccw_ref, s, r, p * qh, qh, h + p,
                                ccw_send, ccw_recv, left)
            sub_cw.start()
            sub_ccw.start()
            subs.append((sub_cw, sub_ccw))
        compute_phase(h)
        for p in range(2):
            sub_cw, sub_ccw = subs[p]
            sub_cw.wait_recv()
            sub_ccw.wait_recv()
            compute_phase(N_DEV - 1, row_off=p * qh, nrows=qh,
                          prefetch_next=(p == 0))
        for sub_cw, sub_ccw in subs:
            sub_cw.wait_send()
            sub_ccw.wait_send()

        amax_out[0, 0] = state["amax"]
        for cp in ys_pending:
            if cp is not None:
                cp.wait()

    return pl.pallas_call(
        body,
        out_shape=[
            jax.ShapeDtypeStruct((M, n_per), jnp.float32),
            jax.ShapeDtypeStruct((1, 1), jnp.float32),
        ],
        in_specs=[
            pl.BlockSpec(memory_space=pl.ANY),
            pl.BlockSpec(memory_space=pl.ANY),
        ],
        out_specs=[
            pl.BlockSpec(memory_space=pl.ANY),
            pl.BlockSpec(memory_space=pltpu.SMEM),
        ],
        scratch_shapes=[
            pltpu.VMEM((2, half, K), jnp.float32),
            pltpu.VMEM((2, half, K), jnp.float32),
            pltpu.VMEM((2, K, bn), jnp.float32),
            pltpu.VMEM((2, half, bn), jnp.float32),
            pltpu.SemaphoreType.DMA((N_DEV,)),
            pltpu.SemaphoreType.DMA((N_DEV,)),
            pltpu.SemaphoreType.DMA((N_DEV,)),
            pltpu.SemaphoreType.DMA((N_DEV,)),
            pltpu.SemaphoreType.REGULAR,
            pltpu.SemaphoreType.REGULAR,
            pltpu.SemaphoreType.DMA((2,)),
            pltpu.SemaphoreType.DMA((2,)),
            pltpu.SemaphoreType.DMA((2,)),
        ],
        compiler_params=pltpu.CompilerParams(
            collective_id=0, vmem_limit_bytes=60 * 2**20),
    )(x, w)


def _global_amax(local_amax):

    def body(a_ref, g_ref, buf_ref, send_sems, recv_sems):
        my = lax.axis_index("i")

        barrier_sem = pltpu.get_barrier_semaphore()
        for off in range(1, N_DEV):
            tgt = lax.rem(my + off, N_DEV)
            pl.semaphore_signal(
                barrier_sem, inc=1,
                device_id=(tgt,), device_id_type=pl.DeviceIdType.MESH,
            )
        pl.semaphore_wait(barrier_sem, N_DEV - 1)

        buf_ref[my] = jnp.full((8, 128), a_ref[0, 0], jnp.float32)

        sends = []
        for off in range(1, N_DEV):
            tgt = lax.rem(my + off, N_DEV)
            rdma = pltpu.make_async_remote_copy(
                src_ref=buf_ref.at[my],
                dst_ref=buf_ref.at[my],
                send_sem=send_sems.at[off - 1],
                recv_sem=recv_sems.at[off - 1],
                device_id=(tgt,),
                device_id_type=pl.DeviceIdType.MESH,
            )
            rdma.start()
            sends.append(rdma)

        for off in range(1, N_DEV):
            src = lax.rem(my + N_DEV - off, N_DEV)
            recv = pltpu.make_async_remote_copy(
                src_ref=buf_ref.at[my],
                dst_ref=buf_ref.at[src],
                send_sem=send_sems.at[off - 1],
                recv_sem=recv_sems.at[off - 1],
                device_id=(src,),
                device_id_type=pl.DeviceIdType.MESH,
            )
            recv.wait_recv()
        for rdma in sends:
            rdma.wait_send()

        g_ref[0, 0] = jnp.max(buf_ref[...])

    return pl.pallas_call(
        body,
        out_shape=jax.ShapeDtypeStruct((1, 1), jnp.float32),
        in_specs=[pl.BlockSpec(memory_space=pltpu.SMEM)],
        out_specs=pl.BlockSpec(memory_space=pltpu.SMEM),
        scratch_shapes=[
            pltpu.VMEM((N_DEV, 8, 128), jnp.float32),
            pltpu.SemaphoreType.DMA((N_DEV - 1,)),
            pltpu.SemaphoreType.DMA((N_DEV - 1,)),
        ],
        compiler_params=pltpu.CompilerParams(collective_id=1),
    )(local_amax)


def _quant_epilogue(y, gmax):
    M, n_per = y.shape
    bm = 1024

    def body(g_ref, y_ref, o_ref):
        g = jnp.maximum(g_ref[0, 0], 1e-30)
        scale = g / 448.0
        inv = 448.0 / g
        q = (y_ref[...] * inv).astype(jnp.float8_e4m3fn)
        o_ref[...] = q.astype(jnp.float32) * scale

    return pl.pallas_call(
        body,
        grid=(M // bm,),
        in_specs=[
            pl.BlockSpec((1, 1), lambda i: (0, 0), memory_space=pltpu.SMEM),
            pl.BlockSpec((bm, n_per), lambda i: (i, 0)),
        ],
        out_specs=pl.BlockSpec((bm, n_per), lambda i: (i, 0)),
        out_shape=jax.ShapeDtypeStruct((M, n_per), jnp.float32),
        compiler_params=pltpu.CompilerParams(vmem_limit_bytes=58 * 2**20),
    )(gmax, y)


def _exchange_quant(y, local_amax):
    M, n_per = y.shape
    bm = 1024

    def body(a_ref, y_ref, o_ref, scale_ref, buf_ref, send_sems, recv_sems):
        i = pl.program_id(0)

        @pl.when(i == 0)
        def _():
            my = lax.axis_index("i")

            barrier_sem = pltpu.get_barrier_semaphore()
            for off in range(1, N_DEV):
                tgt = lax.rem(my + off, N_DEV)
                pl.semaphore_signal(
                    barrier_sem, inc=1,
                    device_id=(tgt,), device_id_type=pl.DeviceIdType.MESH,
                )
            pl.semaphore_wait(barrier_sem, N_DEV - 1)

            buf_ref[my] = jnp.full((8, 128), a_ref[0, 0], jnp.float32)

            sends = []
            for off in range(1, N_DEV):
                tgt = lax.rem(my + off, N_DEV)
                rdma = pltpu.make_async_remote_copy(
                    src_ref=buf_ref.at[my],
                    dst_ref=buf_ref.at[my],
                    send_sem=send_sems.at[off - 1],
                    recv_sem=recv_sems.at[off - 1],
                    device_id=(tgt,),
                    device_id_type=pl.DeviceIdType.MESH,
                )
                rdma.start()
                sends.append(rdma)

            for off in range(1, N_DEV):
                src = lax.rem(my + N_DEV - off, N_DEV)
                recv = pltpu.make_async_remote_copy(
                    src_ref=buf_ref.at[my],
                    dst_ref=buf_ref.at[src],
                    send_sem=send_sems.at[off - 1],
                    recv_sem=recv_sems.at[off - 1],
                    device_id=(src,),
                    device_id_type=pl.DeviceIdType.MESH,
                )
                recv.wait_recv()
            for rdma in sends:
                rdma.wait_send()

            g = jnp.maximum(jnp.max(buf_ref[...]), 1e-30)
            scale_ref[0] = g / 448.0
            scale_ref[1] = 448.0 / g

        q = (y_ref[...] * scale_ref[1]).astype(jnp.float8_e4m3fn)
        o_ref[...] = q.astype(jnp.float32) * scale_ref[0]

    return pl.pallas_call(
        body,
        grid=(M // bm,),
        in_specs=[
            pl.BlockSpec((1, 1), lambda i: (0, 0), memory_space=pltpu.SMEM),
            pl.BlockSpec((bm, n_per), lambda i: (i, 0)),
        ],
        out_specs=pl.BlockSpec((bm, n_per), lambda i: (i, 0)),
        out_shape=jax.ShapeDtypeStruct((M, n_per), jnp.float32),
        scratch_shapes=[
            pltpu.SMEM((2,), jnp.float32),
            pltpu.VMEM((N_DEV, 8, 128), jnp.float32),
            pltpu.SemaphoreType.DMA((N_DEV - 1,)),
            pltpu.SemaphoreType.DMA((N_DEV - 1,)),
        ],
        compiler_params=pltpu.CompilerParams(
            collective_id=1, vmem_limit_bytes=58 * 2**20),
    )(local_amax, y)


def kernel(x, w_mat):
    y, local_amax = _ag_gemm_fused(x, w_mat)
    gmax = _global_amax(local_amax)
    return _quant_epilogue(y, gmax)


# device time: 318343 ns/iter; 1.1176x vs baseline; 1.0130x over previous
import jax
import jax.numpy as jnp
from jax import lax
from jax.experimental import pallas as pl
from jax.experimental.pallas import tpu as pltpu

N_DEV = 4


def _ring_allgather(x):
    m_per, k = x.shape
    M = N_DEV * m_per

    def body(x_hbm, out_hbm, comm_ref, send_sems, recv_sems, local_sem):
        my = lax.axis_index("i")
        left = lax.rem(my + N_DEV - 1, N_DEV)
        right = lax.rem(my + 1, N_DEV)

        barrier_sem = pltpu.get_barrier_semaphore()
        for nbr in [left, right]:
            pl.semaphore_signal(
                barrier_sem, inc=1,
                device_id=(nbr,), device_id_type=pl.DeviceIdType.MESH,
            )
        pl.semaphore_wait(barrier_sem, 2)

        cp = pltpu.make_async_copy(x_hbm, comm_ref.at[0], local_sem)
        cp.start()
        cp.wait()
        cp = pltpu.make_async_copy(
            comm_ref.at[0], out_hbm.at[pl.ds(my * m_per, m_per)], local_sem
        )
        cp.start()
        cp.wait()

        for h in range(N_DEV - 1):
            s, r = h % 2, (h + 1) % 2
            rdma = pltpu.make_async_remote_copy(
                src_ref=comm_ref.at[s],
                dst_ref=comm_ref.at[r],
                send_sem=send_sems.at[s],
                recv_sem=recv_sems.at[r],
                device_id=(right,),
                device_id_type=pl.DeviceIdType.MESH,
            )
            rdma.start()
            rdma.wait()
            origin = lax.rem(my + N_DEV - 1 - h, N_DEV)
            cp = pltpu.make_async_copy(
                comm_ref.at[r], out_hbm.at[pl.ds(origin * m_per, m_per)], local_sem
            )
            cp.start()
            cp.wait()

    return pl.pallas_call(
        body,
        out_shape=jax.ShapeDtypeStruct((M, k), x.dtype),
        in_specs=[pl.BlockSpec(memory_space=pl.ANY)],
        out_specs=pl.BlockSpec(memory_space=pl.ANY),
        scratch_shapes=[
            pltpu.VMEM((2, m_per, k), x.dtype),
            pltpu.SemaphoreType.DMA((2,)),
            pltpu.SemaphoreType.DMA((2,)),
            pltpu.SemaphoreType.DMA,
        ],
        compiler_params=pltpu.CompilerParams(collective_id=0),
    )(x)


def _gemm_relu_amax(x_full, w):
    M, K = x_full.shape
    _, n_per = w.shape
    bm, bn = 512, 512

    def body(x_ref, w_ref, y_ref, amax_out, acc_ref):
        i, j = pl.program_id(0), pl.program_id(1)

        @pl.when((i == 0) & (j == 0))
        def _():
            acc_ref[0, 0] = 0.0

        z = jnp.dot(x_ref[...], w_ref[...], preferred_element_type=jnp.float32)
        z = jnp.maximum(z, 0.0)
        y_ref[...] = z
        acc_ref[0, 0] = jnp.maximum(acc_ref[0, 0], jnp.max(z))
        amax_out[0, 0] = acc_ref[0, 0]

    return pl.pallas_call(
        body,
        grid=(M // bm, n_per // bn),
        in_specs=[
            pl.BlockSpec((bm, K), lambda i, j: (i, 0)),
            pl.BlockSpec((K, bn), lambda i, j: (0, j)),
        ],
        out_specs=[
            pl.BlockSpec((bm, bn), lambda i, j: (i, j)),
            pl.BlockSpec((1, 1), lambda i, j: (0, 0), memory_space=pltpu.SMEM),
        ],
        out_shape=[
            jax.ShapeDtypeStruct((M, n_per), jnp.float32),
            jax.ShapeDtypeStruct((1, 1), jnp.float32),
        ],
        scratch_shapes=[pltpu.SMEM((1, 1), jnp.float32)],
        compiler_params=pltpu.CompilerParams(vmem_limit_bytes=58 * 2**20),
    )(x_full, w)


def _ag_gemm_fused(x, w):
    m_per, K = x.shape
    _, n_per = w.shape
    M = N_DEV * m_per
    half = m_per // 2
    bn = 512
    n_tiles = n_per // bn

    def body(x_hbm, w_hbm, y_hbm, gmax_out,
             cw_ref, ccw_ref, wt_ref, ys_ref, amax_buf,
             cw_send, cw_recv, ccw_send, ccw_recv,
             credit_cw, credit_ccw, wsem, ysem, xsem,
             amax_send, amax_recv):
        my = lax.axis_index("i")
        left = lax.rem(my + N_DEV - 1, N_DEV)
        right = lax.rem(my + 1, N_DEV)

        cp_top = pltpu.make_async_copy(
            x_hbm.at[pl.ds(0, half)], cw_ref.at[0], xsem.at[0])
        cp_bot = pltpu.make_async_copy(
            x_hbm.at[pl.ds(half, half)], ccw_ref.at[0], xsem.at[1])
        cp_top.start()
        cp_bot.start()

        wt_pending = {}

        def load_wtile(j, slot):
            cp = pltpu.make_async_copy(
                w_hbm.at[:, pl.ds(j * bn, bn)], wt_ref.at[slot],
                wsem.at[slot])
            cp.start()
            wt_pending[slot] = cp

        load_wtile(0, 0)

        barrier_sem = pltpu.get_barrier_semaphore()
        for nbr in [left, right]:
            pl.semaphore_signal(
                barrier_sem, inc=1,
                device_id=(nbr,), device_id_type=pl.DeviceIdType.MESH,
            )
        pl.semaphore_wait(barrier_sem, 2)

        state = {"amax": jnp.float32(0.0), "store": 0}
        ys_pending = [None, None]

        def store_y(z, row, j, nrows):
            slot = state["store"] % 2
            state["store"] += 1
            if ys_pending[slot] is not None:
                ys_pending[slot].wait()
            ys_ref[slot, :nrows] = z
            cp = pltpu.make_async_copy(
                ys_ref.at[slot, pl.ds(0, nrows)],
                y_hbm.at[pl.ds(row, nrows), pl.ds(j * bn, bn)],
                ysem.at[slot])
            cp.start()
            ys_pending[slot] = cp

        def compute_phase(h, row_off=0, nrows=half, prefetch_next=True):
            s = h % 2
            row_a = lax.rem(my + N_DEV - h, N_DEV) * m_per + row_off
            row_b = lax.rem(my + h, N_DEV) * m_per + half + row_off
            for j in range(n_tiles):
                slot = j % 2
                wt_pending.pop(slot).wait()
                if j + 1 < n_tiles:
                    load_wtile(j + 1, 1 - slot)
                elif prefetch_next:
                    load_wtile(0, 1 - slot)
                za = jnp.dot(cw_ref[s, row_off:row_off + nrows],
                             wt_ref[slot],
                             preferred_element_type=jnp.float32)
                za = jnp.maximum(za, 0.0)
                state["amax"] = jnp.maximum(state["amax"], jnp.max(za))
                store_y(za, row_a, j, nrows)
                zb = jnp.dot(ccw_ref[s, row_off:row_off + nrows],
                             wt_ref[slot],
                             preferred_element_type=jnp.float32)
                zb = jnp.maximum(zb, 0.0)
                state["amax"] = jnp.maximum(state["amax"], jnp.max(zb))
                store_y(zb, row_b, j, nrows)

        def make_rdma(ref, s, r, row_off, nrows, sem_idx, send, recv, tgt):
            return pltpu.make_async_remote_copy(
                src_ref=ref.at[s, pl.ds(row_off, nrows)],
                dst_ref=ref.at[r, pl.ds(row_off, nrows)],
                send_sem=send.at[sem_idx], recv_sem=recv.at[sem_idx],
                device_id=(tgt,), device_id_type=pl.DeviceIdType.MESH)

        for h in range(N_DEV - 2):
            s, r = h % 2, (h + 1) % 2
            if h > 0:
                pl.semaphore_wait(credit_cw, 1)
                pl.semaphore_wait(credit_ccw, 1)
                rdma_cw = make_rdma(cw_ref, s, r, 0, half, h,
                                    cw_send, cw_recv, right)
                rdma_ccw = make_rdma(ccw_ref, s, r, 0, half, h,
                                     ccw_send, ccw_recv, left)
            else:
                rdma_cw = pltpu.make_async_remote_copy(
                    src_ref=x_hbm.at[pl.ds(0, half)],
                    dst_ref=cw_ref.at[1, pl.ds(0, half)],
                    send_sem=cw_send.at[0], recv_sem=cw_recv.at[0],
                    device_id=(right,), device_id_type=pl.DeviceIdType.MESH)
                rdma_ccw = pltpu.make_async_remote_copy(
                    src_ref=x_hbm.at[pl.ds(half, half)],
                    dst_ref=ccw_ref.at[1, pl.ds(0, half)],
                    send_sem=ccw_send.at[0], recv_sem=ccw_recv.at[0],
                    device_id=(left,), device_id_type=pl.DeviceIdType.MESH)
            rdma_cw.start()
            rdma_ccw.start()
            if h == 0:
                cp_top.wait()
                cp_bot.wait()
            compute_phase(h)
            rdma_cw.wait_send()
            rdma_ccw.wait_send()
            pl.semaphore_signal(
                credit_cw, inc=1,
                device_id=(left,), device_id_type=pl.DeviceIdType.MESH)
            pl.semaphore_signal(
                credit_ccw, inc=1,
                device_id=(right,), device_id_type=pl.DeviceIdType.MESH)
            rdma_cw.wait_recv()
            rdma_ccw.wait_recv()

        h = N_DEV - 2
        s, r = h % 2, (h + 1) % 2
        n_sub = 4
        qs = half // n_sub
        pl.semaphore_wait(credit_cw, 1)
        pl.semaphore_wait(credit_ccw, 1)
        subs = []
        for p in range(n_sub):
            sub_cw = make_rdma(cw_ref, s, r, p * qs, qs, h + p,
                               cw_send, cw_recv, right)
            sub_ccw = make_rdma(ccw_ref, s, r, p * qs, qs, h + p,
                                ccw_send, ccw_recv, left)
            sub_cw.start()
            sub_ccw.start()
            subs.append((sub_cw, sub_ccw))
        compute_phase(h)
        for p in range(n_sub):
            sub_cw, sub_ccw = subs[p]
            sub_cw.wait_recv()
            sub_ccw.wait_recv()
            compute_phase(N_DEV - 1, row_off=p * qs, nrows=qs,
                          prefetch_next=(p < n_sub - 1))

        amax_buf[my] = jnp.full((8, 128), state["amax"], jnp.float32)
        amax_sends = []
        for off in range(1, N_DEV):
            tgt = lax.rem(my + off, N_DEV)
            rdma = pltpu.make_async_remote_copy(
                src_ref=amax_buf.at[my], dst_ref=amax_buf.at[my],
                send_sem=amax_send.at[off - 1],
                recv_sem=amax_recv.at[off - 1],
                device_id=(tgt,), device_id_type=pl.DeviceIdType.MESH)
            rdma.start()
            amax_sends.append(rdma)

        for sub_cw, sub_ccw in subs:
            sub_cw.wait_send()
            sub_ccw.wait_send()

        for off in range(1, N_DEV):
            src = lax.rem(my + N_DEV - off, N_DEV)
            recv = pltpu.make_async_remote_copy(
                src_ref=amax_buf.at[my],
                dst_ref=amax_buf.at[src],
                send_sem=amax_send.at[off - 1],
                recv_sem=amax_recv.at[off - 1],
                device_id=(src,), device_id_type=pl.DeviceIdType.MESH)
            recv.wait_recv()
        for rdma in amax_sends:
            rdma.wait_send()

        gmax_out[0, 0] = jnp.maximum(jnp.max(amax_buf[...]), 1e-30)
        for cp in ys_pending:
            if cp is not None:
                cp.wait()

    return pl.pallas_call(
        body,
        out_shape=[
            jax.ShapeDtypeStruct((M, n_per), jnp.float32),
            jax.ShapeDtypeStruct((1, 1), jnp.float32),
        ],
        in_specs=[
            pl.BlockSpec(memory_space=pl.ANY),
            pl.BlockSpec(memory_space=pl.ANY),
        ],
        out_specs=[
            pl.BlockSpec(memory_space=pl.ANY),
            pl.BlockSpec(memory_space=pltpu.SMEM),
        ],
        scratch_shapes=[
            pltpu.VMEM((2, half, K), jnp.float32),
            pltpu.VMEM((2, half, K), jnp.float32),
            pltpu.VMEM((2, K, bn), jnp.float32),
            pltpu.VMEM((2, half, bn), jnp.float32),
            pltpu.VMEM((N_DEV, 8, 128), jnp.float32),
            pltpu.SemaphoreType.DMA((6,)),
            pltpu.SemaphoreType.DMA((6,)),
            pltpu.SemaphoreType.DMA((6,)),
            pltpu.SemaphoreType.DMA((6,)),
            pltpu.SemaphoreType.REGULAR,
            pltpu.SemaphoreType.REGULAR,
            pltpu.SemaphoreType.DMA((2,)),
            pltpu.SemaphoreType.DMA((2,)),
            pltpu.SemaphoreType.DMA((2,)),
            pltpu.SemaphoreType.DMA((N_DEV - 1,)),
            pltpu.SemaphoreType.DMA((N_DEV - 1,)),
        ],
        compiler_params=pltpu.CompilerParams(
            collective_id=0, vmem_limit_bytes=60 * 2**20),
    )(x, w)


def _global_amax(local_amax):

    def body(a_ref, g_ref, buf_ref, send_sems, recv_sems):
        my = lax.axis_index("i")

        barrier_sem = pltpu.get_barrier_semaphore()
        for off in range(1, N_DEV):
            tgt = lax.rem(my + off, N_DEV)
            pl.semaphore_signal(
                barrier_sem, inc=1,
                device_id=(tgt,), device_id_type=pl.DeviceIdType.MESH,
            )
        pl.semaphore_wait(barrier_sem, N_DEV - 1)

        buf_ref[my] = jnp.full((8, 128), a_ref[0, 0], jnp.float32)

        sends = []
        for off in range(1, N_DEV):
            tgt = lax.rem(my + off, N_DEV)
            rdma = pltpu.make_async_remote_copy(
                src_ref=buf_ref.at[my],
                dst_ref=buf_ref.at[my],
                send_sem=send_sems.at[off - 1],
                recv_sem=recv_sems.at[off - 1],
                device_id=(tgt,),
                device_id_type=pl.DeviceIdType.MESH,
            )
            rdma.start()
            sends.append(rdma)

        for off in range(1, N_DEV):
            src = lax.rem(my + N_DEV - off, N_DEV)
            recv = pltpu.make_async_remote_copy(
                src_ref=buf_ref.at[my],
                dst_ref=buf_ref.at[src],
                send_sem=send_sems.at[off - 1],
                recv_sem=recv_sems.at[off - 1],
                device_id=(src,),
                device_id_type=pl.DeviceIdType.MESH,
            )
            recv.wait_recv()
        for rdma in sends:
            rdma.wait_send()

        g_ref[0, 0] = jnp.max(buf_ref[...])

    return pl.pallas_call(
        body,
        out_shape=jax.ShapeDtypeStruct((1, 1), jnp.float32),
        in_specs=[pl.BlockSpec(memory_space=pltpu.SMEM)],
        out_specs=pl.BlockSpec(memory_space=pltpu.SMEM),
        scratch_shapes=[
            pltpu.VMEM((N_DEV, 8, 128), jnp.float32),
            pltpu.SemaphoreType.DMA((N_DEV - 1,)),
            pltpu.SemaphoreType.DMA((N_DEV - 1,)),
        ],
        compiler_params=pltpu.CompilerParams(collective_id=1),
    )(local_amax)


def _quant_epilogue(y, gmax):
    M, n_per = y.shape
    bm = 1024

    def body(g_ref, y_ref, o_ref):
        g = jnp.maximum(g_ref[0, 0], 1e-30)
        scale = g / 448.0
        inv = 448.0 / g
        q = (y_ref[...] * inv).astype(jnp.float8_e4m3fn)
        o_ref[...] = q.astype(jnp.float32) * scale

    return pl.pallas_call(
        body,
        grid=(M // bm,),
        in_specs=[
            pl.BlockSpec((1, 1), lambda i: (0, 0), memory_space=pltpu.SMEM),
            pl.BlockSpec((bm, n_per), lambda i: (i, 0)),
        ],
        out_specs=pl.BlockSpec((bm, n_per), lambda i: (i, 0)),
        out_shape=jax.ShapeDtypeStruct((M, n_per), jnp.float32),
        compiler_params=pltpu.CompilerParams(vmem_limit_bytes=58 * 2**20),
    )(gmax, y)


def _exchange_quant(y, local_amax):
    M, n_per = y.shape
    bm = 1024

    def body(a_ref, y_ref, o_ref, scale_ref, buf_ref, send_sems, recv_sems):
        i = pl.program_id(0)

        @pl.when(i == 0)
        def _():
            my = lax.axis_index("i")

            barrier_sem = pltpu.get_barrier_semaphore()
            for off in range(1, N_DEV):
                tgt = lax.rem(my + off, N_DEV)
                pl.semaphore_signal(
                    barrier_sem, inc=1,
                    device_id=(tgt,), device_id_type=pl.DeviceIdType.MESH,
                )
            pl.semaphore_wait(barrier_sem, N_DEV - 1)

            buf_ref[my] = jnp.full((8, 128), a_ref[0, 0], jnp.float32)

            sends = []
            for off in range(1, N_DEV):
                tgt = lax.rem(my + off, N_DEV)
                rdma = pltpu.make_async_remote_copy(
                    src_ref=buf_ref.at[my],
                    dst_ref=buf_ref.at[my],
                    send_sem=send_sems.at[off - 1],
                    recv_sem=recv_sems.at[off - 1],
                    device_id=(tgt,),
                    device_id_type=pl.DeviceIdType.MESH,
                )
                rdma.start()
                sends.append(rdma)

            for off in range(1, N_DEV):
                src = lax.rem(my + N_DEV - off, N_DEV)
                recv = pltpu.make_async_remote_copy(
                    src_ref=buf_ref.at[my],
                    dst_ref=buf_ref.at[src],
                    send_sem=send_sems.at[off - 1],
                    recv_sem=recv_sems.at[off - 1],
                    device_id=(src,),
                    device_id_type=pl.DeviceIdType.MESH,
                )
                recv.wait_recv()
            for rdma in sends:
                rdma.wait_send()

            g = jnp.maximum(jnp.max(buf_ref[...]), 1e-30)
            scale_ref[0] = g / 448.0
            scale_ref[1] = 448.0 / g

        q = (y_ref[...] * scale_ref[1]).astype(jnp.float8_e4m3fn)
        o_ref[...] = q.astype(jnp.float32) * scale_ref[0]

    return pl.pallas_call(
        body,
        grid=(M // bm,),
        in_specs=[
            pl.BlockSpec((1, 1), lambda i: (0, 0), memory_space=pltpu.SMEM),
            pl.BlockSpec((bm, n_per), lambda i: (i, 0)),
        ],
        out_specs=pl.BlockSpec((bm, n_per), lambda i: (i, 0)),
        out_shape=jax.ShapeDtypeStruct((M, n_per), jnp.float32),
        scratch_shapes=[
            pltpu.SMEM((2,), jnp.float32),
            pltpu.VMEM((N_DEV, 8, 128), jnp.float32),
            pltpu.SemaphoreType.DMA((N_DEV - 1,)),
            pltpu.SemaphoreType.DMA((N_DEV - 1,)),
        ],
        compiler_params=pltpu.CompilerParams(
            collective_id=1, vmem_limit_bytes=58 * 2**20),
    )(local_amax, y)


def kernel(x, w_mat):
    y, gmax = _ag_gemm_fused(x, w_mat)
    return _quant_epilogue(y, gmax)
